# baseline (speedup 1.0000x reference)
import jax
import jax.numpy as jnp
from jax import lax
from jax.experimental import pallas as pl
from jax.experimental.pallas import tpu as pltpu
from jax.experimental.pallas import tpu_sc as plsc

N = 100000
D = 128
S = 256
NC = 2
NS = 16
DH = D // NC
C = 128
SUB = 2
MC = C * SUB
NB = 4
L = 16

R = 512
TC_BLOCKS = 48
Q = R * TC_BLOCKS

SC_N = N - Q
FULL_MEGA = SC_N // MC
NITER = FULL_MEGA // NS
REM = FULL_MEGA - NITER * NS
TAIL = SC_N - FULL_MEGA * MC
assert TAIL == 160 and Q % 8 == 0
TAIL2 = TAIL - C


def _drain(descs):
    for d in descs:
        d.wait()


def _sc_body(data_hbm, ids_hbm, out_hbm,
          data2, ids3, ids_t, data_t, stage, acc, red_rows, red_ids, *sems):
    c = lax.axis_index("c")
    s = lax.axis_index("s")
    col0 = c * DH
    sem_ld = sems[:NB]
    sem_sc = sems[NB:]

    for r in range(L):
        for j in range(DH // L):
            stage[r, pl.ds(j * L, L)] = jnp.zeros((L,), jnp.float32)
    pltpu.sync_copy(stage, acc.at[pl.ds(s * L, L)])

    @pl.when(s < 2)
    def _():
        pltpu.sync_copy(stage, acc.at[pl.ds((NS + s) * L, L)])

    lane = lax.iota(jnp.int32, L)
    dummy = lane * 0 + S
    for q in range(64 // L):
        red_ids[pl.ds(q * L, L)] = dummy
    plsc.subcore_barrier()

    def issue_loads(i, b):
        base = Q + (s + i * NS) * MC
        descs = [pltpu.async_copy(
            data_hbm.at[pl.ds(base, MC), pl.ds(col0, DH)],
            data2.at[b], sem_ld[b])]
        for j in range(SUB):
            descs.append(pltpu.async_copy(
                ids_hbm.at[pl.ds(base + j * C, C)], ids3.at[b, j], sem_ld[b]))
        return descs

    lane0 = lane == 0

    def process_chunk(i, b):
        for j in range(SUB):
            f16 = ids3[b, j, pl.ds(0, L)]
            l16 = ids3[b, j, pl.ds(C - L, L)]
            single = jnp.min(f16) == jnp.max(l16)
            k = i * SUB + j

            @pl.when(single)
            def _():
                zero = jnp.zeros((L,), jnp.float32)

                init = (zero,) * (4 * (DH // L))

                def red4(r, carry):
                    accs = list(carry)
                    for u in range(4):
                        for g in range(DH // L):
                            accs[g * 4 + u] = accs[g * 4 + u] + data2[
                                b, j * C + 4 * r + u, pl.ds(g * L, L)]
                    return tuple(accs)

                accs = lax.fori_loop(0, C // 4, red4, init)
                for g in range(DH // L):
                    tot = (accs[g * 4] + accs[g * 4 + 1]
                           + accs[g * 4 + 2] + accs[g * 4 + 3])
                    red_rows[k, pl.ds(g * L, L)] = tot
                plsc.store_scatter(red_ids, [lane * 0 + k], f16, mask=lane0)

            @pl.when(jnp.logical_not(single))
            def _():
                pltpu.sync_copy(data2.at[b, pl.ds(j * C, C)],
                                acc.at[ids3.at[b, j]], add=True)

    ld_pend = [None] * NB
    for k in range(NB - 1):
        ld_pend[k] = issue_loads(k, k)
    for i in range(NITER):
        b = i % NB
        nk = i + NB - 1
        if nk < NITER:
            ld_pend[nk % NB] = issue_loads(nk, nk % NB)
        _drain(ld_pend[b])
        process_chunk(i, b)
    pltpu.sync_copy(red_rows, acc.at[red_ids], add=True)

    @pl.when(s < REM)
    def _():
        base = Q + (NITER * NS + s) * MC
        pltpu.sync_copy(data_hbm.at[pl.ds(base, MC), pl.ds(col0, DH)],
                        data2.at[0])
        for j in range(SUB):
            pltpu.sync_copy(ids_hbm.at[pl.ds(base + j * C, C)], ids3.at[0, j])
        for j in range(SUB):
            pltpu.sync_copy(data2.at[0, pl.ds(j * C, C)],
                            acc.at[ids3.at[0, j]], add=True)

    @pl.when(s == NS - 1)
    def _():
        base = Q + FULL_MEGA * MC
        pltpu.sync_copy(data_hbm.at[pl.ds(base, C), pl.ds(col0, DH)],
                        data2.at[0, pl.ds(0, C)])
        pltpu.sync_copy(ids_hbm.at[pl.ds(base, C)], ids3.at[0, 0])
        pltpu.sync_copy(data2.at[0, pl.ds(0, C)], acc.at[ids3.at[0, 0]],
                        add=True)
        base2 = base + C
        pltpu.sync_copy(data_hbm.at[pl.ds(base2, TAIL2), pl.ds(col0, DH)],
                        data_t)
        pltpu.sync_copy(ids_hbm.at[pl.ds(base2, TAIL2)], ids_t)
        pltpu.sync_copy(data_t, acc.at[ids_t], add=True)

    plsc.subcore_barrier()

    pltpu.sync_copy(acc.at[pl.ds(s * L, L)], stage)
    pltpu.sync_copy(stage, out_hbm.at[pl.ds(s * L, L), pl.ds(col0, DH)])


def _tc_body(ids_ref, data_ref, out_ref):
    i = pl.program_id(0)
    ids_row = ids_ref[0, 0, :]
    seg = lax.broadcasted_iota(jnp.int32, (S, R), 0)
    one_hot = (seg == ids_row[None, :]).astype(jnp.bfloat16)
    blk = data_ref[...]
    hi = blk.astype(jnp.bfloat16)
    lo = (blk - hi.astype(jnp.float32)).astype(jnp.bfloat16)
    part = (jnp.dot(one_hot, hi, preferred_element_type=jnp.float32)
            + jnp.dot(one_hot, lo, preferred_element_type=jnp.float32))

    @pl.when(i == 0)
    def _():
        out_ref[...] = part

    @pl.when(i > 0)
    def _():
        out_ref[...] += part


def kernel(data, segment_ids):
    mesh = plsc.VectorSubcoreMesh(core_axis_name="c", subcore_axis_name="s",
                                  num_cores=NC, num_subcores=NS)
    run = pl.kernel(
        _sc_body,
        out_type=jax.ShapeDtypeStruct((S, D), jnp.float32),
        mesh=mesh,
        scratch_types=[
            pltpu.VMEM((NB, MC, DH), jnp.float32),
            pltpu.VMEM((NB, SUB, C), jnp.int32),
            pltpu.VMEM((TAIL2,), jnp.int32),
            pltpu.VMEM((TAIL2, DH), jnp.float32),
            pltpu.VMEM((L, DH), jnp.float32),
            pltpu.VMEM_SHARED((S + 2 * L, DH), jnp.float32),
            pltpu.VMEM((64, DH), jnp.float32),
            pltpu.VMEM((64,), jnp.int32),
        ] + [pltpu.SemaphoreType.DMA] * (2 * NB),
        compiler_params=pltpu.CompilerParams(use_tc_tiling_on_sc=False,
                                            needs_layout_passes=False),
    )
    ids32 = segment_ids.astype(jnp.int32)
    sc_out = run(data, ids32)

    tc_ids = ids32[:Q].reshape(TC_BLOCKS, 1, R)
    tc_out = pl.pallas_call(
        _tc_body,
        grid=(TC_BLOCKS,),
        in_specs=[
            pl.BlockSpec((1, 1, R), lambda i: (i, 0, 0)),
            pl.BlockSpec((R, D), lambda i: (i, 0)),
        ],
        out_specs=pl.BlockSpec((S, D), lambda i: (0, 0)),
        out_shape=jax.ShapeDtypeStruct((S, D), jnp.float32),
    )(tc_ids, data)

    return sc_out + tc_out


# speedup vs baseline: 1.0357x; 1.0357x over previous
import jax
import jax.numpy as jnp
from jax import lax
from jax.experimental import pallas as pl
from jax.experimental.pallas import tpu as pltpu
from jax.experimental.pallas import tpu_sc as plsc

N = 100000
D = 128
S = 256
NC = 2
NS = 16
DH = D // NC
C = 128
SUB = 2
MC = C * SUB
NB = 4
L = 16

R = 512
TC_BLOCKS = 48
Q = R * TC_BLOCKS

SC_N = N - Q
FULL_MEGA = SC_N // MC
NITER = FULL_MEGA // NS
REM = FULL_MEGA - NITER * NS
TAIL = SC_N - FULL_MEGA * MC
assert TAIL == 160 and Q % 8 == 0
TAIL2 = TAIL - C


def _drain(descs):
    for d in descs:
        d.wait()


def _sc_body(data_hbm, ids_hbm, out_hbm,
          data2, ids3, ids_t, data_t, stage, acc, *sems):
    c = lax.axis_index("c")
    s = lax.axis_index("s")
    col0 = c * DH
    sem_ld = sems[:NB]
    sem_sc = sems[NB:]

    for r in range(L):
        for j in range(DH // L):
            stage[r, pl.ds(j * L, L)] = jnp.zeros((L,), jnp.float32)
    pltpu.sync_copy(stage, acc.at[pl.ds(s * L, L)])

    @pl.when(s < 2)
    def _():
        pltpu.sync_copy(stage, acc.at[pl.ds((NS + s) * L, L)])

    plsc.subcore_barrier()

    def issue_loads(i, b):
        base = Q + (s + i * NS) * MC
        descs = [pltpu.async_copy(
            data_hbm.at[pl.ds(base, MC), pl.ds(col0, DH)],
            data2.at[b], sem_ld[b])]
        for j in range(SUB):
            descs.append(pltpu.async_copy(
                ids_hbm.at[pl.ds(base + j * C, C)], ids3.at[b, j], sem_ld[b]))
        return descs

    def issue_scatters(b):
        return [pltpu.async_copy(
            data2.at[b, pl.ds(j * C, C)], acc.at[ids3.at[b, j]],
            sem_sc[b], add=True) for j in range(SUB)]

    ld_pend = [None] * NB
    sc_pend = [None] * NB
    for k in range(NB - 1):
        ld_pend[k] = issue_loads(k, k)
    for i in range(NITER):
        b = i % NB
        nk = i + NB - 1
        if nk < NITER:
            nb_ = nk % NB
            if sc_pend[nb_] is not None:
                _drain(sc_pend[nb_])
            ld_pend[nb_] = issue_loads(nk, nb_)
        _drain(ld_pend[b])
        sc_pend[b] = issue_scatters(b)
    for b in range(NB):
        if sc_pend[b] is not None:
            _drain(sc_pend[b])

    @pl.when(s < REM)
    def _():
        base = Q + (NITER * NS + s) * MC
        pltpu.sync_copy(data_hbm.at[pl.ds(base, MC), pl.ds(col0, DH)],
                        data2.at[0])
        for j in range(SUB):
            pltpu.sync_copy(ids_hbm.at[pl.ds(base + j * C, C)], ids3.at[0, j])
        for j in range(SUB):
            pltpu.sync_copy(data2.at[0, pl.ds(j * C, C)],
                            acc.at[ids3.at[0, j]], add=True)

    @pl.when(s == NS - 1)
    def _():
        base = Q + FULL_MEGA * MC
        pltpu.sync_copy(data_hbm.at[pl.ds(base, C), pl.ds(col0, DH)],
                        data2.at[0, pl.ds(0, C)])
        pltpu.sync_copy(ids_hbm.at[pl.ds(base, C)], ids3.at[0, 0])
        pltpu.sync_copy(data2.at[0, pl.ds(0, C)], acc.at[ids3.at[0, 0]],
                        add=True)
        base2 = base + C
        pltpu.sync_copy(data_hbm.at[pl.ds(base2, TAIL2), pl.ds(col0, DH)],
                        data_t)
        pltpu.sync_copy(ids_hbm.at[pl.ds(base2, TAIL2)], ids_t)
        pltpu.sync_copy(data_t, acc.at[ids_t], add=True)

    plsc.subcore_barrier()

    pltpu.sync_copy(acc.at[pl.ds(s * L, L)], stage)
    pltpu.sync_copy(stage, out_hbm.at[pl.ds(s * L, L), pl.ds(col0, DH)])


def _tc_body(ids_ref, data_ref, out_ref):
    i = pl.program_id(0)
    ids_row = ids_ref[0, 0, :]
    seg = lax.broadcasted_iota(jnp.int32, (S, R), 0)
    one_hot = (seg == ids_row[None, :]).astype(jnp.float32)
    part = jnp.dot(one_hot, data_ref[...],
                   preferred_element_type=jnp.float32)

    @pl.when(i == 0)
    def _():
        out_ref[...] = part

    @pl.when(i > 0)
    def _():
        out_ref[...] += part


def kernel(data, segment_ids):
    mesh = plsc.VectorSubcoreMesh(core_axis_name="c", subcore_axis_name="s",
                                  num_cores=NC, num_subcores=NS)
    run = pl.kernel(
        _sc_body,
        out_type=jax.ShapeDtypeStruct((S, D), jnp.float32),
        mesh=mesh,
        scratch_types=[
            pltpu.VMEM((NB, MC, DH), jnp.float32),
            pltpu.VMEM((NB, SUB, C), jnp.int32),
            pltpu.VMEM((TAIL2,), jnp.int32),
            pltpu.VMEM((TAIL2, DH), jnp.float32),
            pltpu.VMEM((L, DH), jnp.float32),
            pltpu.VMEM_SHARED((S + 2 * L, DH), jnp.float32),
        ] + [pltpu.SemaphoreType.DMA] * (2 * NB),
        compiler_params=pltpu.CompilerParams(use_tc_tiling_on_sc=False,
                                            needs_layout_passes=False),
    )
    ids32 = segment_ids.astype(jnp.int32)
    sc_out = run(data, ids32)

    tc_ids = ids32[:Q].reshape(TC_BLOCKS, 1, R)
    tc_out = pl.pallas_call(
        _tc_body,
        grid=(TC_BLOCKS,),
        in_specs=[
            pl.BlockSpec((1, 1, R), lambda i: (i, 0, 0)),
            pl.BlockSpec((R, D), lambda i: (i, 0)),
        ],
        out_specs=pl.BlockSpec((S, D), lambda i: (0, 0)),
        out_shape=jax.ShapeDtypeStruct((S, D), jnp.float32),
    )(tc_ids, data)

    return sc_out + tc_out
